# baseline (speedup 1.0000x reference)
import functools
import math

import jax
import jax.numpy as jnp
from jax import lax
from jax.experimental import pallas as pl
from jax.experimental.pallas import tpu as pltpu
from jax.experimental.pallas import tpu_sc as plsc

N = 10000
E = 320000
DX = 128
DE = 16
P1 = 64
P2 = 32
B = 64
EPS = 1e-5

NC = 2
NS = 16
NW = NC * NS
EW_T = E // NW
CK = 100
NCHUNK = EW_T // CK
FIRE = 10
ROUNDS = NCHUNK // FIRE
NPS = N // NS

ET = 1280
_SC_MESH = dict(core_axis_name="c", subcore_axis_name="s", num_cores=NC,
                num_subcores=NS)
_SC_PARAMS = pltpu.CompilerParams(use_tc_tiling_on_sc=False)


def _leaky(v):
    return jnp.where(v > 0, v, 0.01 * v)


def _lin0_body(x_ref, w_ref, b_ref, o_ref):
    o_ref[...] = _leaky(
        jnp.dot(x_ref[...], w_ref[...], preferred_element_type=jnp.float32)
        + b_ref[...])


def _msg_body(ea_ref, g_ref, w1_ref, b1_ref, w2_ref, b2m_ref, msg_ref):
    ew = _leaky(
        jnp.dot(ea_ref[...], w1_ref[...], preferred_element_type=jnp.float32)
        + b1_ref[...])
    c = jnp.dot(ew, w2_ref[...], preferred_element_type=jnp.float32)
    g = g_ref[...]
    red = jnp.sum(c.reshape(ET, P2, P2) * g[:, :, None], axis=1)
    msg_ref[...] = red + jnp.dot(g, b2m_ref[...],
                                 preferred_element_type=jnp.float32)


def _update_body(ap_ref, cnt_ref, out_ref, wroot_ref, wih_ref, whh_ref,
                 bc_ref, bih_ref, bhh_ref, new_ref):
    cnt = jnp.maximum(cnt_ref[0] + cnt_ref[1], 1.0)
    aggr = (ap_ref[0] + ap_ref[1]) / cnt
    o = out_ref[...]
    m = _leaky(
        jnp.dot(o, wroot_ref[...], preferred_element_type=jnp.float32)
        + aggr + bc_ref[...])
    gi = jnp.dot(m, wih_ref[...], preferred_element_type=jnp.float32) \
        + bih_ref[...]
    gh = jnp.dot(o, whh_ref[...], preferred_element_type=jnp.float32) \
        + bhh_ref[...]
    r = jax.nn.sigmoid(gi[:, :P2] + gh[:, :P2])
    z = jax.nn.sigmoid(gi[:, P2:2 * P2] + gh[:, P2:2 * P2])
    n = jnp.tanh(gi[:, 2 * P2:] + r * gh[:, 2 * P2:])
    new_ref[...] = (1.0 - z) * n + z * o


def _s2s_body(out_ref, batch_ref, lwih_ref, lwhh_ref, lbi_ref, lbh_ref,
              w1_ref, b1_ref, w2_ref, b2_ref, wf_ref, bf_ref, y_ref):
    o = out_ref[...]
    onehot = (batch_ref[...] ==
              lax.broadcasted_iota(jnp.int32, (N, B), 1)).astype(jnp.float32)
    q_star = jnp.zeros((B, 2 * P2), jnp.float32)
    hl = jnp.zeros((B, P2), jnp.float32)
    cl = jnp.zeros((B, P2), jnp.float32)
    for _ in range(3):
        gates = (jnp.dot(q_star, lwih_ref[...],
                         preferred_element_type=jnp.float32) + lbi_ref[...]
                 + jnp.dot(hl, lwhh_ref[...],
                           preferred_element_type=jnp.float32) + lbh_ref[...])
        i_ = jax.nn.sigmoid(gates[:, :P2])
        f_ = jax.nn.sigmoid(gates[:, P2:2 * P2])
        g_ = jnp.tanh(gates[:, 2 * P2:3 * P2])
        o_ = jax.nn.sigmoid(gates[:, 3 * P2:])
        cl = f_ * cl + i_ * g_
        hl = o_ * jnp.tanh(cl)
        q = hl
        qn = jnp.dot(onehot, q, preferred_element_type=jnp.float32)
        e = jnp.sum(o * qn, axis=1, keepdims=True)
        col = jnp.where(onehot > 0, e, -1e30)
        emax = jnp.max(col, axis=0, keepdims=True)
        emax_n = jnp.sum(onehot * emax, axis=1, keepdims=True)
        a = jnp.exp(e - emax_n)
        asum = jnp.sum(onehot * a, axis=0, keepdims=True)
        asum_n = jnp.sum(onehot * asum, axis=1, keepdims=True)
        a = a / asum_n
        rvec = lax.dot_general(onehot * a, o, (((0,), (0,)), ((), ())),
                               preferred_element_type=jnp.float32)
        q_star = jnp.concatenate([q, rvec], axis=1)
    y = _leaky(jnp.dot(q_star, w1_ref[...],
                       preferred_element_type=jnp.float32) + b1_ref[...])
    y = _leaky(jnp.dot(y, w2_ref[...],
                       preferred_element_type=jnp.float32) + b2_ref[...])
    y_ref[...] = jnp.dot(y, wf_ref[...],
                         preferred_element_type=jnp.float32) + bf_ref[...]


def _sc_gather_body(table_hbm, idx_hbm, out_hbm, idx_v, rows_v, sem):
    wid = lax.axis_index("s") * NC + lax.axis_index("c")
    pltpu.sync_copy(idx_hbm.at[wid], idx_v)
    base = wid * EW_T

    def round_body(r, carry):
        rbase = r * FIRE
        for b in range(FIRE):
            pltpu.async_copy(table_hbm.at[idx_v.at[rbase + b]],
                             rows_v.at[pl.ds(b * CK, CK)], sem)
        for b in range(FIRE):
            pltpu.make_async_copy(table_hbm.at[idx_v.at[rbase + b]],
                                  rows_v.at[pl.ds(b * CK, CK)], sem).wait()
        pltpu.sync_copy(rows_v,
                        out_hbm.at[pl.ds(base + rbase * CK, FIRE * CK)])
        return carry

    lax.fori_loop(0, ROUNDS, round_body, 0)


def _sc_scatter_body(rows_hbm, idx_hbm, zeros_hbm, out_hbm,
                     idx_v, rows_v, accum_sh, sem):
    c = lax.axis_index("c")
    s = lax.axis_index("s")
    wid = s * NC + c
    pltpu.sync_copy(idx_hbm.at[wid], idx_v)
    pltpu.sync_copy(zeros_hbm.at[pl.ds(s * NPS, NPS)],
                    accum_sh.at[pl.ds(s * NPS, NPS)])
    plsc.subcore_barrier()
    base = wid * EW_T

    def round_body(r, carry):
        rbase = r * FIRE
        pltpu.sync_copy(rows_hbm.at[pl.ds(base + rbase * CK, FIRE * CK)],
                        rows_v)
        for b in range(FIRE):
            pltpu.async_copy(rows_v.at[pl.ds(b * CK, CK)],
                             accum_sh.at[idx_v.at[rbase + b]], sem, add=True)
        for b in range(FIRE):
            pltpu.make_async_copy(rows_v.at[pl.ds(b * CK, CK)],
                                  accum_sh.at[idx_v.at[rbase + b]],
                                  sem).wait()
        return carry

    lax.fori_loop(0, ROUNDS, round_body, 0)
    plsc.subcore_barrier()
    pltpu.sync_copy(accum_sh.at[pl.ds(s * NPS, NPS)],
                    out_hbm.at[c, pl.ds(s * NPS, NPS)])


def _sc_ones_scatter_body(ones_hbm, idx_hbm, zeros_hbm, out_hbm,
                          idx_v, rows_v, accum_sh, sem):
    c = lax.axis_index("c")
    s = lax.axis_index("s")
    wid = s * NC + c
    pltpu.sync_copy(idx_hbm.at[wid], idx_v)
    pltpu.sync_copy(zeros_hbm.at[pl.ds(s * NPS, NPS)],
                    accum_sh.at[pl.ds(s * NPS, NPS)])
    pltpu.sync_copy(ones_hbm, rows_v)
    plsc.subcore_barrier()

    def round_body(r, carry):
        rbase = r * FIRE
        for b in range(FIRE):
            pltpu.async_copy(rows_v.at[pl.ds(b * CK, CK)],
                             accum_sh.at[idx_v.at[rbase + b]], sem, add=True)
        for b in range(FIRE):
            pltpu.make_async_copy(rows_v.at[pl.ds(b * CK, CK)],
                                  accum_sh.at[idx_v.at[rbase + b]],
                                  sem).wait()
        return carry

    lax.fori_loop(0, ROUNDS, round_body, 0)
    plsc.subcore_barrier()
    pltpu.sync_copy(accum_sh.at[pl.ds(s * NPS, NPS)],
                    out_hbm.at[c, pl.ds(s * NPS, NPS)])


def _make_sc_gather():
    return pl.kernel(
        _sc_gather_body,
        out_type=jax.ShapeDtypeStruct((E, P2), jnp.float32),
        mesh=plsc.VectorSubcoreMesh(**_SC_MESH),
        compiler_params=_SC_PARAMS,
        scratch_types=[
            pltpu.VMEM((NCHUNK, CK), jnp.int32),
            pltpu.VMEM((FIRE * CK, P2), jnp.float32),
            pltpu.SemaphoreType.DMA,
        ])


def _make_sc_scatter(body):
    return pl.kernel(
        body,
        out_type=jax.ShapeDtypeStruct((NC, N, P2), jnp.float32),
        mesh=plsc.VectorSubcoreMesh(**_SC_MESH),
        compiler_params=_SC_PARAMS,
        scratch_types=[
            pltpu.VMEM((NCHUNK, CK), jnp.int32),
            pltpu.VMEM((FIRE * CK, P2), jnp.float32),
            pltpu.VMEM_SHARED((N, P2), jnp.float32),
            pltpu.SemaphoreType.DMA,
        ])


def kernel(x, edge_index, edge_attr, batch, W0, g0, b0, We1, ge1, be1,
           We2, ge2, be2, Wroot, bconv, Wih, Whh, bih, bhh,
           lWih, lWhh, lbih, lbhh, W1, b1, W2, b2, Wf, bf):
    f32 = jnp.float32
    cbn = 1.0 / math.sqrt(1.0 + EPS)

    W0f = W0 * (cbn * g0)[None, :]
    W1f = We1 * (cbn * ge1)[None, :]
    W2f = We2 * (cbn * ge2)[None, :]
    B2m = be2.reshape(P2, P2)

    src_r = edge_index[0].reshape(NW, NCHUNK, CK).astype(jnp.int32)
    dst_r = edge_index[1].reshape(NW, NCHUNK, CK).astype(jnp.int32)
    zeros_n = jnp.zeros((N, P2), f32)
    ones_rows = jnp.ones((FIRE * CK, P2), f32)

    lin0 = pl.pallas_call(
        _lin0_body, out_shape=jax.ShapeDtypeStruct((N, P2), f32))
    out = lin0(x, W0f, b0.reshape(1, P2))

    cnt_parts = _make_sc_scatter(_sc_ones_scatter_body)(
        ones_rows, dst_r, zeros_n)

    msg_call = pl.pallas_call(
        _msg_body,
        grid=(E // ET,),
        in_specs=[
            pl.BlockSpec((ET, DE), lambda i: (i, 0)),
            pl.BlockSpec((ET, P2), lambda i: (i, 0)),
            pl.BlockSpec((DE, P1), lambda i: (0, 0)),
            pl.BlockSpec((1, P1), lambda i: (0, 0)),
            pl.BlockSpec((P1, P2 * P2), lambda i: (0, 0)),
            pl.BlockSpec((P2, P2), lambda i: (0, 0)),
        ],
        out_specs=pl.BlockSpec((ET, P2), lambda i: (i, 0)),
        out_shape=jax.ShapeDtypeStruct((E, P2), f32))

    update_call = pl.pallas_call(
        _update_body, out_shape=jax.ShapeDtypeStruct((N, P2), f32))

    gather = _make_sc_gather()
    scatter = _make_sc_scatter(_sc_scatter_body)

    for _ in range(3):
        g = gather(out, src_r)
        msg = msg_call(edge_attr, g, W1f, be1.reshape(1, P1), W2f, B2m)
        aggr_parts = scatter(msg, dst_r, zeros_n)
        out = update_call(aggr_parts, cnt_parts, out, Wroot, Wih, Whh,
                          bconv.reshape(1, P2), bih.reshape(1, 3 * P2),
                          bhh.reshape(1, 3 * P2))

    s2s_call = pl.pallas_call(
        _s2s_body, out_shape=jax.ShapeDtypeStruct((B, 1), f32))
    y = s2s_call(out, batch.reshape(N, 1).astype(jnp.int32),
                 lWih, lWhh, lbih.reshape(1, 4 * P2), lbhh.reshape(1, 4 * P2),
                 W1, b1.reshape(1, P2), W2, b2.reshape(1, P2 // 2),
                 Wf, bf.reshape(1, 1))
    return y.reshape(B)
